# Initial kernel scaffold; baseline (speedup 1.0000x reference)
import jax
import jax.numpy as jnp
from jax.experimental import pallas as pl


def kernel(x, region_ids, pos_table):
    raise NotImplementedError("write your pallas kernel here")


# speedup vs baseline: 1.1630x; 1.1630x over previous
import functools

import jax
import jax.numpy as jnp
from jax import lax
from jax.experimental import pallas as pl
from jax.experimental.pallas import tpu as pltpu
from jax.experimental.pallas import tpu_sc as plsc

D_MODEL = 1024
SEQ = 8192
NUM_CORES = 2
NUM_SUBCORES = 16
NUM_WORKERS = NUM_CORES * NUM_SUBCORES
ROWS_PER_WORKER = SEQ // NUM_WORKERS
CHUNK = 64
NUM_CHUNKS = ROWS_PER_WORKER // CHUNK

_SC_MESH = plsc.VectorSubcoreMesh(core_axis_name="c", subcore_axis_name="s")


@functools.partial(
    pl.kernel,
    mesh=_SC_MESH,
    out_type=jax.ShapeDtypeStruct((SEQ, D_MODEL), jnp.float32),
    scratch_types=[
        pltpu.VMEM((CHUNK,), jnp.int32),
        pltpu.VMEM((CHUNK, D_MODEL), jnp.float32),
        pltpu.SemaphoreType.DMA,
    ],
)
def _gather_sc(table_hbm, idx_hbm, out_hbm, idx_v, rows_v, sem):
    wid = lax.axis_index("s") * NUM_CORES + lax.axis_index("c")
    base = wid * ROWS_PER_WORKER

    def body(c, carry):
        off = base + c * CHUNK
        pltpu.sync_copy(idx_hbm.at[pl.ds(off, CHUNK)], idx_v)
        pltpu.async_copy(table_hbm.at[idx_v], rows_v, sem).wait()
        pltpu.sync_copy(rows_v, out_hbm.at[pl.ds(off, CHUNK)])
        return carry

    lax.fori_loop(0, NUM_CHUNKS, body, 0)


def _add_body(x_ref, p_ref, o_ref):
    o_ref[...] = x_ref[...] + p_ref[...][None, :, :]


def _add_tc(x, pos_embed):
    b, s, d = x.shape
    s_blk = 256
    return pl.pallas_call(
        _add_body,
        grid=(s // s_blk,),
        in_specs=[
            pl.BlockSpec((b, s_blk, d), lambda i: (0, i, 0)),
            pl.BlockSpec((s_blk, d), lambda i: (i, 0)),
        ],
        out_specs=pl.BlockSpec((b, s_blk, d), lambda i: (0, i, 0)),
        out_shape=jax.ShapeDtypeStruct(x.shape, x.dtype),
    )(x, pos_embed)


def kernel(x, region_ids, pos_table):
    ids = region_ids.astype(jnp.int32)
    pos_embed = _gather_sc(pos_table, ids)
    return _add_tc(x, pos_embed)
